# Initial kernel scaffold; baseline (speedup 1.0000x reference)
import jax
import jax.numpy as jnp
from jax.experimental import pallas as pl


def kernel(x, W_in, b_in, codebook, W_out, b_out):
    raise NotImplementedError("write your pallas kernel here")


# speedup vs baseline: 1.3718x; 1.3718x over previous
import functools

import jax
import jax.numpy as jnp
from jax import lax
from jax.experimental import pallas as pl
from jax.experimental.pallas import tpu as pltpu
from jax.experimental.pallas import tpu_sc as plsc

B, T, D, Dc, K = 8, 1024, 768, 64, 8192
N = B * T


TM1 = 256
G1 = N // TM1


def _vq_body(x_ref, wi_ref, bi_ref, cbt_ref, z_ref, idx_ref):
    x = x_ref[...]
    z = jnp.dot(x, wi_ref[...], preferred_element_type=jnp.float32) + bi_ref[...]
    z_ref[...] = z
    cbt = cbt_ref[...]
    s = jnp.dot(z, cbt, preferred_element_type=jnp.float32)
    e2 = jnp.sum(cbt * cbt, axis=0)
    d = e2[None, :] - 2.0 * s
    idx = jnp.argmin(d, axis=1).astype(jnp.int32)
    idx_ref[...] = idx.reshape(1, 1, TM1)


def _project_quantize(xr, W_in, b_in2, cbT):
    return pl.pallas_call(
        _vq_body,
        grid=(G1,),
        in_specs=[
            pl.BlockSpec((TM1, D), lambda i: (i, 0)),
            pl.BlockSpec((D, Dc), lambda i: (0, 0)),
            pl.BlockSpec((1, Dc), lambda i: (0, 0)),
            pl.BlockSpec((Dc, K), lambda i: (0, 0)),
        ],
        out_specs=[
            pl.BlockSpec((TM1, Dc), lambda i: (i, 0)),
            pl.BlockSpec((1, 1, TM1), lambda i: (i, 0, 0)),
        ],
        out_shape=[
            jax.ShapeDtypeStruct((N, Dc), jnp.float32),
            jax.ShapeDtypeStruct((G1, 1, TM1), jnp.int32),
        ],
    )(xr, W_in, b_in2, cbT)


NC, NS, L = 2, 16, 16
NW = NC * NS
BPW = N // NW
DP = 128


def _gather_body(cb_hbm, idx_hbm, out_hbm, idx_v, rows_v, sem):
    wid = lax.axis_index("s") * NC + lax.axis_index("c")
    base = wid * BPW
    pltpu.sync_copy(idx_hbm.at[pl.ds(base, BPW)], idx_v)
    pltpu.async_copy(cb_hbm.at[idx_v], rows_v, sem).wait()
    pltpu.sync_copy(rows_v, out_hbm.at[pl.ds(base, BPW)])


_codebook_gather = functools.partial(
    pl.kernel,
    mesh=plsc.VectorSubcoreMesh(core_axis_name="c", subcore_axis_name="s"),
    out_type=jax.ShapeDtypeStruct((N, DP), jnp.float32),
    scratch_types=[
        pltpu.VMEM((BPW,), jnp.int32),
        pltpu.VMEM((BPW, DP), jnp.float32),
        pltpu.SemaphoreType.DMA,
    ],
)(_gather_body)


TM2 = 512
G2 = N // TM2


def _out_body(q_ref, z_ref, wo_ref, bo_ref, qf_ref, acc_ref):
    q = q_ref[:, :Dc]
    z = z_ref[...]
    diff = q - z
    q_st = z + diff
    qf_ref[...] = jnp.dot(q_st, wo_ref[...], preferred_element_type=jnp.float32) + bo_ref[...]

    @pl.when(pl.program_id(0) == 0)
    def _():
        acc_ref[0, 0] = 0.0

    acc_ref[0, 0] += jnp.sum(diff * diff)


def _project_out(q, z, W_out, b_out2):
    return pl.pallas_call(
        _out_body,
        grid=(G2,),
        in_specs=[
            pl.BlockSpec((TM2, DP), lambda i: (i, 0)),
            pl.BlockSpec((TM2, Dc), lambda i: (i, 0)),
            pl.BlockSpec((Dc, D), lambda i: (0, 0)),
            pl.BlockSpec((1, D), lambda i: (0, 0)),
        ],
        out_specs=[
            pl.BlockSpec((TM2, D), lambda i: (i, 0)),
            pl.BlockSpec(memory_space=pltpu.SMEM),
        ],
        out_shape=[
            jax.ShapeDtypeStruct((N, D), jnp.float32),
            jax.ShapeDtypeStruct((1, 1), jnp.float32),
        ],
    )(q, z, W_out, b_out2)


def kernel(x, W_in, b_in, codebook, W_out, b_out):
    xr = x.reshape(N, D)
    z, idx3 = _project_quantize(xr, W_in, b_in.reshape(1, Dc), codebook.T)
    indices = idx3.reshape(N)
    cb_pad = jnp.pad(codebook, ((0, 0), (0, DP - Dc)))
    q = _codebook_gather(cb_pad, indices)
    qf, acc = _project_out(q, z, W_out, b_out.reshape(1, D))
    vq_loss = acc[0, 0] * (1.25 / (N * Dc))
    return (qf.reshape(B, T, D), indices.reshape(B, T), vq_loss)


# speedup vs baseline: 1.6469x; 1.2005x over previous
import functools

import jax
import jax.numpy as jnp
from jax import lax
from jax.experimental import pallas as pl
from jax.experimental.pallas import tpu as pltpu
from jax.experimental.pallas import tpu_sc as plsc

B, T, D, Dc, K = 8, 1024, 768, 64, 8192
N = B * T


TM1 = 512
G1 = N // TM1
DA = 72


def _vq_body(x_ref, wi_ref, bi_ref, cbt_ref, z_ref, idx_ref, caug_ref):
    @pl.when(pl.program_id(0) == 0)
    def _():
        cbt = cbt_ref[...]
        caug_ref[0:Dc, :] = -2.0 * cbt
        caug_ref[Dc:Dc + 1, :] = jnp.sum(cbt * cbt, axis=0, keepdims=True)
        caug_ref[Dc + 1:DA, :] = jnp.zeros((DA - Dc - 1, K), jnp.float32)

    x = x_ref[...]
    z = jnp.dot(x, wi_ref[...], preferred_element_type=jnp.float32) + bi_ref[...]
    z_ref[...] = z
    z_aug = jnp.concatenate(
        [z, jnp.ones((TM1, 1), jnp.float32), jnp.zeros((TM1, DA - Dc - 1), jnp.float32)],
        axis=1)
    d = jnp.dot(z_aug, caug_ref[...], preferred_element_type=jnp.float32)
    idx = jnp.argmin(d, axis=1).astype(jnp.int32)
    idx_ref[...] = idx.reshape(1, 1, TM1)


def _project_quantize(xr, W_in, b_in2, cbT):
    return pl.pallas_call(
        _vq_body,
        grid=(G1,),
        in_specs=[
            pl.BlockSpec((TM1, D), lambda i: (i, 0)),
            pl.BlockSpec((D, Dc), lambda i: (0, 0)),
            pl.BlockSpec((1, Dc), lambda i: (0, 0)),
            pl.BlockSpec((Dc, K), lambda i: (0, 0)),
        ],
        out_specs=[
            pl.BlockSpec((TM1, Dc), lambda i: (i, 0)),
            pl.BlockSpec((1, 1, TM1), lambda i: (i, 0, 0)),
        ],
        out_shape=[
            jax.ShapeDtypeStruct((N, Dc), jnp.float32),
            jax.ShapeDtypeStruct((G1, 1, TM1), jnp.int32),
        ],
        scratch_shapes=[pltpu.VMEM((DA, K), jnp.float32)],
    )(xr, W_in, b_in2, cbT)


NC, NS, L = 2, 16, 16
NW = NC * NS
BPW = N // NW
DP = 128


def _gather_body(cb_hbm, idx_hbm, out_hbm, idx_v, rows_v, sem):
    wid = lax.axis_index("s") * NC + lax.axis_index("c")
    base = wid * BPW
    pltpu.sync_copy(idx_hbm.at[pl.ds(base, BPW)], idx_v)
    pltpu.async_copy(cb_hbm.at[idx_v], rows_v, sem).wait()
    pltpu.sync_copy(rows_v, out_hbm.at[pl.ds(base, BPW)])


_codebook_gather = functools.partial(
    pl.kernel,
    mesh=plsc.VectorSubcoreMesh(core_axis_name="c", subcore_axis_name="s"),
    out_type=jax.ShapeDtypeStruct((N, DP), jnp.float32),
    scratch_types=[
        pltpu.VMEM((BPW,), jnp.int32),
        pltpu.VMEM((BPW, DP), jnp.float32),
        pltpu.SemaphoreType.DMA,
    ],
)(_gather_body)


TM2 = 512
G2 = N // TM2


def _out_body(q_ref, z_ref, wo_ref, bo_ref, qf_ref, acc_ref):
    q = q_ref[:, :Dc]
    z = z_ref[...]
    diff = q - z
    q_st = z + diff
    qf_ref[...] = jnp.dot(q_st, wo_ref[...], preferred_element_type=jnp.float32) + bo_ref[...]

    @pl.when(pl.program_id(0) == 0)
    def _():
        acc_ref[0, 0] = 0.0

    acc_ref[0, 0] += jnp.sum(diff * diff)


def _project_out(q, z, W_out, b_out2):
    return pl.pallas_call(
        _out_body,
        grid=(G2,),
        in_specs=[
            pl.BlockSpec((TM2, DP), lambda i: (i, 0)),
            pl.BlockSpec((TM2, Dc), lambda i: (i, 0)),
            pl.BlockSpec((Dc, D), lambda i: (0, 0)),
            pl.BlockSpec((1, D), lambda i: (0, 0)),
        ],
        out_specs=[
            pl.BlockSpec((TM2, D), lambda i: (i, 0)),
            pl.BlockSpec(memory_space=pltpu.SMEM),
        ],
        out_shape=[
            jax.ShapeDtypeStruct((N, D), jnp.float32),
            jax.ShapeDtypeStruct((1, 1), jnp.float32),
        ],
    )(q, z, W_out, b_out2)


def kernel(x, W_in, b_in, codebook, W_out, b_out):
    xr = x.reshape(N, D)
    z, idx3 = _project_quantize(xr, W_in, b_in.reshape(1, Dc), codebook.T)
    indices = idx3.reshape(N)
    cb_pad = jnp.pad(codebook, ((0, 0), (0, DP - Dc)))
    q = _codebook_gather(cb_pad, indices)
    qf, acc = _project_out(q, z, W_out, b_out.reshape(1, D))
    vq_loss = acc[0, 0] * (1.25 / (N * Dc))
    return (qf.reshape(B, T, D), indices.reshape(B, T), vq_loss)
